# baseline (speedup 1.0000x reference)
import functools

import jax
import jax.numpy as jnp
from jax import lax
from jax.experimental import pallas as pl
from jax.experimental.pallas import tpu as pltpu
from jax.experimental.pallas import tpu_sc as plsc

S, D, H, HD = 2048, 2048, 16, 128
E, TOPK, FF = 8, 2, 1024
BM = 256
BN = 512
BT = 256
KB = 512
A = S * TOPK
PADDED = A + E * BT
NT = PADDED // BT
NW = 32


def _ln_qkv_body(x_ref, g_ref, b_ref, wq_ref, bq_ref, wk_ref, bk_ref,
                 wv_ref, bv_ref, q_ref, k_ref, v_ref):
    x = x_ref[...]
    m = jnp.mean(x, axis=-1, keepdims=True)
    var = jnp.mean((x - m) ** 2, axis=-1, keepdims=True)
    h = (x - m) / jnp.sqrt(var + 1e-5) * g_ref[...] + b_ref[...]
    q_ref[...] = jnp.dot(h, wq_ref[...], preferred_element_type=jnp.float32) + bq_ref[...]
    k_ref[...] = jnp.dot(h, wk_ref[...], preferred_element_type=jnp.float32) + bk_ref[...]
    v_ref[...] = jnp.dot(h, wv_ref[...], preferred_element_type=jnp.float32) + bv_ref[...]


def _ln_qkv(x, g, b, wq, bq, wk, bk, wv, bv):
    row = pl.BlockSpec((BM, D), lambda i, j: (i, 0))
    wsp = pl.BlockSpec((D, BN), lambda i, j: (0, j))
    bsp = pl.BlockSpec((1, BN), lambda i, j: (0, j))
    vec = pl.BlockSpec((1, D), lambda i, j: (0, 0))
    out = pl.BlockSpec((BM, BN), lambda i, j: (i, j))
    return pl.pallas_call(
        _ln_qkv_body,
        grid=(S // BM, D // BN),
        in_specs=[row, vec, vec, wsp, bsp, wsp, bsp, wsp, bsp],
        out_specs=[out, out, out],
        out_shape=[jax.ShapeDtypeStruct((S, D), jnp.float32)] * 3,
    )(x, g.reshape(1, D), b.reshape(1, D), wq, bq.reshape(1, D),
      wk, bk.reshape(1, D), wv, bv.reshape(1, D))


def _rot_half(x):
    return jnp.concatenate([-x[:, HD // 2:], x[:, : HD // 2]], axis=1)


NHP = 4


def _attn_body(q_ref, k_ref, v_ref, c_ref, s_ref, o_ref):
    i = pl.program_id(1)
    cq = c_ref[pl.ds(i * BM, BM), :]
    sq = s_ref[pl.ds(i * BM, BM), :]
    qq = q_ref[...]
    qs = [((qq[:, t * HD:(t + 1) * HD] * cq
            + _rot_half(qq[:, t * HD:(t + 1) * HD]) * sq).astype(jnp.bfloat16))
          for t in range(NHP)]

    def step(j, carry):
        kk = k_ref[pl.ds(j * KB, KB), :]
        ck = c_ref[pl.ds(j * KB, KB), :]
        sk = s_ref[pl.ds(j * KB, KB), :]
        vv = v_ref[pl.ds(j * KB, KB), :]
        rows = i * BM + lax.broadcasted_iota(jnp.int32, (BM, KB), 0)
        cols = j * KB + lax.broadcasted_iota(jnp.int32, (BM, KB), 1)
        causal = cols <= rows
        out = []
        for t in range(NHP):
            acc, m, l = carry[3 * t], carry[3 * t + 1], carry[3 * t + 2]
            kx = kk[:, t * HD:(t + 1) * HD]
            kj = (kx * ck + _rot_half(kx) * sk).astype(jnp.bfloat16)
            vj = vv[:, t * HD:(t + 1) * HD].astype(jnp.bfloat16)
            s = lax.dot_general(qs[t], kj, (((1,), (1,)), ((), ())),
                                preferred_element_type=jnp.float32)
            s = s * (1.0 / jnp.sqrt(jnp.float32(HD)))
            s = jnp.where(causal, s, jnp.float32(-1e9))
            m_new = jnp.maximum(m, jnp.max(s, axis=1, keepdims=True))
            p = jnp.exp(s - m_new)
            corr = jnp.exp(m - m_new)
            l = l * corr + jnp.sum(p, axis=1, keepdims=True)
            acc = acc * corr + jnp.dot(p.astype(jnp.bfloat16), vj,
                                       preferred_element_type=jnp.float32)
            out += [acc, m_new, l]
        return tuple(out)

    z = jnp.zeros((BM, HD), jnp.float32)
    mneg = jnp.full((BM, 1), -1e30, jnp.float32)
    zl = jnp.zeros((BM, 1), jnp.float32)
    res = lax.fori_loop(0, (i * BM) // KB + 1, step, (z, mneg, zl) * NHP)
    o_ref[...] = jnp.concatenate(
        [res[3 * t] / res[3 * t + 2] for t in range(NHP)], axis=1)


def _attention(q, k, v, cos, sin):
    qsp = pl.BlockSpec((BM, NHP * HD), lambda h, i: (i, h))
    ksp = pl.BlockSpec((S, NHP * HD), lambda h, i: (0, h))
    csp = pl.BlockSpec((S, HD), lambda h, i: (0, 0))
    return pl.pallas_call(
        _attn_body,
        grid=(H // NHP, S // BM),
        in_specs=[qsp, ksp, ksp, csp, csp],
        out_specs=qsp,
        out_shape=jax.ShapeDtypeStruct((S, D), jnp.float32),
    )(q, k, v, cos, sin)


def _oproj_body(a_ref, w_ref, b_ref, r_ref, o_ref):
    o_ref[...] = (jnp.dot(a_ref[...], w_ref[...], preferred_element_type=jnp.float32)
                  + b_ref[...] + r_ref[...])


def _oproj_res(attn, wo, bo, res):
    return pl.pallas_call(
        _oproj_body,
        grid=(S // BM, D // BN),
        in_specs=[pl.BlockSpec((BM, D), lambda i, j: (i, 0)),
                  pl.BlockSpec((D, BN), lambda i, j: (0, j)),
                  pl.BlockSpec((1, BN), lambda i, j: (0, j)),
                  pl.BlockSpec((BM, BN), lambda i, j: (i, j))],
        out_specs=pl.BlockSpec((BM, BN), lambda i, j: (i, j)),
        out_shape=jax.ShapeDtypeStruct((S, D), jnp.float32),
    )(attn, wo, bo.reshape(1, D), res)


def _gate_body(x_ref, g_ref, b_ref, wg_ref, h_ref, mi_ref, mf_ref):
    x = x_ref[...]
    m = jnp.mean(x, axis=-1, keepdims=True)
    var = jnp.mean((x - m) ** 2, axis=-1, keepdims=True)
    h = (x - m) / jnp.sqrt(var + 1e-5) * g_ref[...] + b_ref[...]
    h_ref[...] = h
    lg = jnp.dot(h, wg_ref[...], preferred_element_type=jnp.float32)
    col = lax.broadcasted_iota(jnp.int32, lg.shape, 1)
    neg = jnp.float32(-1e30)
    lg = jnp.where(col < E, lg, neg)
    m1 = jnp.max(lg, axis=1, keepdims=True)
    i1 = jnp.min(jnp.where(lg >= m1, col, 128), axis=1, keepdims=True)
    lg2 = jnp.where(col == i1, neg, lg)
    m2 = jnp.max(lg2, axis=1, keepdims=True)
    i2 = jnp.min(jnp.where(lg2 >= m2, col, 128), axis=1, keepdims=True)
    w1 = 1.0 / (1.0 + jnp.exp(m2 - m1))
    w2 = 1.0 - w1
    mi_ref[...] = jnp.where(col == 0, i1, jnp.where(col == 1, i2, 0))
    mf_ref[...] = jnp.where(col == 0, w1, jnp.where(col == 1, w2, 0.0))


def _ln_gate(x, g, b, wg_pad):
    row = pl.BlockSpec((BM, D), lambda i: (i, 0))
    vec = pl.BlockSpec((1, D), lambda i: (0, 0))
    meta = pl.BlockSpec((BM, 128), lambda i: (i, 0))
    return pl.pallas_call(
        _gate_body,
        grid=(S // BM,),
        in_specs=[row, vec, vec, pl.BlockSpec((D, 128), lambda i: (0, 0))],
        out_specs=[row, meta, meta],
        out_shape=[jax.ShapeDtypeStruct((S, D), jnp.float32),
                   jax.ShapeDtypeStruct((S, 128), jnp.int32),
                   jax.ShapeDtypeStruct((S, 128), jnp.float32)],
    )(x, g.reshape(1, D), b.reshape(1, D), wg_pad)


def _sc_gather(table, idx, n_rows, chunk=16, nbuf=3):
    per_w = n_rows // NW
    n_chunks = per_w // chunk
    mesh = plsc.VectorSubcoreMesh(core_axis_name="c", subcore_axis_name="s")

    @functools.partial(
        pl.kernel, mesh=mesh,
        out_type=jax.ShapeDtypeStruct((n_rows, D), jnp.float32),
        scratch_types=[pltpu.VMEM((per_w,), jnp.int32),
                       pltpu.VMEM((nbuf, chunk, D), jnp.float32),
                       [pltpu.SemaphoreType.DMA] * nbuf,
                       [pltpu.SemaphoreType.DMA] * nbuf],
    )
    def k(table_hbm, idx_hbm, out_hbm, idx_v, rows_v, gsem, osem):
        wid = lax.axis_index("s") * 2 + lax.axis_index("c")
        base = wid * per_w
        pltpu.sync_copy(idx_hbm.at[pl.ds(base, per_w)], idx_v)
        gd = [None] * n_chunks
        od = [None] * n_chunks

        def start_gather(c):
            b = c % nbuf
            gd[c] = pltpu.async_copy(
                table_hbm.at[idx_v.at[pl.ds(c * chunk, chunk)]], rows_v.at[b], gsem[b])

        for c in range(min(nbuf, n_chunks)):
            start_gather(c)
        for c in range(n_chunks):
            b = c % nbuf
            gd[c].wait()
            od[c] = pltpu.async_copy(
                rows_v.at[b], out_hbm.at[pl.ds(base + c * chunk, chunk)], osem[b])
            if c + nbuf < n_chunks:
                od[c].wait()
                start_gather(c + nbuf)
        for c in range(max(0, n_chunks - nbuf), n_chunks):
            od[c].wait()

    return k(table, idx)


def _ffn_body(te_ref, used_ref, xs_ref, w1_ref, b1_ref, w2_ref, b2_ref,
              ws_ref, o_ref):
    i = pl.program_id(0)

    @pl.when(used_ref[i] > 0)
    def _compute():
        x = xs_ref[...].astype(jnp.bfloat16)
        h = jnp.dot(x, w1_ref[0].astype(jnp.bfloat16),
                    preferred_element_type=jnp.float32) + b1_ref[0]
        h = h * 0.5 * (1.0 + lax.erf(h * (1.0 / jnp.sqrt(jnp.float32(2.0)))))
        o = jnp.dot(h.astype(jnp.bfloat16), w2_ref[0].astype(jnp.bfloat16),
                    preferred_element_type=jnp.float32) + b2_ref[0]
        o_ref[...] = o * ws_ref[...]


def _grouped_ffn(xs, tile_expert, tile_used, w1, b1, w2, b2, w_slot):
    grid_spec = pltpu.PrefetchScalarGridSpec(
        num_scalar_prefetch=2,
        grid=(NT,),
        in_specs=[
            pl.BlockSpec((BT, D), lambda i, te, tu: (i, 0)),
            pl.BlockSpec((1, D, FF), lambda i, te, tu: (te[i], 0, 0)),
            pl.BlockSpec((1, 1, FF), lambda i, te, tu: (te[i], 0, 0)),
            pl.BlockSpec((1, FF, D), lambda i, te, tu: (te[i], 0, 0)),
            pl.BlockSpec((1, 1, D), lambda i, te, tu: (te[i], 0, 0)),
            pl.BlockSpec((BT, 1), lambda i, te, tu: (i, 0)),
        ],
        out_specs=pl.BlockSpec((BT, D), lambda i, te, tu: (i, 0)),
    )
    return pl.pallas_call(
        _ffn_body,
        grid_spec=grid_spec,
        out_shape=jax.ShapeDtypeStruct((PADDED, D), jnp.float32),
    )(tile_expert, tile_used, xs, w1,
      b1.reshape(E, 1, FF), w2, b2.reshape(E, 1, D),
      w_slot.reshape(PADDED, 1))


def _combine_body(r_ref, a_ref, b_ref, o_ref):
    o_ref[...] = r_ref[...] + a_ref[...] + b_ref[...]


def _combine(res, r1, r2):
    blk = pl.BlockSpec((BM, BN), lambda i, j: (i, j))
    return pl.pallas_call(
        _combine_body,
        grid=(S // BM, D // BN),
        in_specs=[blk, blk, blk],
        out_specs=blk,
        out_shape=jax.ShapeDtypeStruct((S, D), jnp.float32),
    )(res, r1, r2)


def kernel(hidden_states, wq, bq, wk, bk, wv, bv, wo, bo,
           ln1_g, ln1_b, ln2_g, ln2_b, w_gate, w1, b1, w2, b2):
    x = hidden_states.reshape(S, D)

    pos = jnp.arange(S, dtype=jnp.float32)
    inv_freq = 1.0 / (10000.0 ** (jnp.arange(0, HD, 2, dtype=jnp.float32) / HD))
    freqs = pos[:, None] * inv_freq[None, :]
    emb = jnp.concatenate((freqs, freqs), axis=-1)
    cos = jnp.cos(emb)
    sin = jnp.sin(emb)

    q, k, v = _ln_qkv(x, ln1_g, ln1_b, wq, bq, wk, bk, wv, bv)
    attn = _attention(q, k, v, cos, sin)
    x2 = _oproj_res(attn, wo, bo, x)

    wg_pad = jnp.pad(w_gate, ((0, 0), (0, 128 - E)))
    h2, mi, mf = _ln_gate(x2, ln2_g, ln2_b, wg_pad)
    topi = mi[:, :TOPK]
    wts = mf[:, :TOPK]

    ef = topi.reshape(-1)
    onehot = (ef[:, None] == jnp.arange(E, dtype=ef.dtype)[None, :]).astype(jnp.int32)
    counts = jnp.sum(onehot, axis=0)
    pc = ((counts + BT - 1) // BT) * BT
    ends_p = jnp.cumsum(pc)
    starts_p = ends_p - pc
    rank = jnp.take_along_axis(jnp.cumsum(onehot, axis=0), ef[:, None], axis=1)[:, 0] - 1
    slot_of_a = (starts_p[ef] + rank).astype(jnp.int32)
    s1 = slot_of_a[0::2]
    s2 = slot_of_a[1::2]
    tok = jnp.arange(A, dtype=jnp.int32) // TOPK
    ri = (jnp.arange(PADDED, dtype=jnp.int32) % S).at[slot_of_a].set(tok)
    w_slot = jnp.zeros((PADDED,), jnp.float32).at[slot_of_a].set(wts.reshape(-1))
    tile_start = jnp.arange(NT, dtype=jnp.int32) * BT
    tile_used = (tile_start < ends_p[-1]).astype(jnp.int32)
    tile_expert = jnp.minimum(
        jnp.searchsorted(ends_p, tile_start, side="right"), E - 1).astype(jnp.int32)

    xs = _sc_gather(h2, ri, PADDED)
    eout = _grouped_ffn(xs, tile_expert, tile_used, w1, b1, w2, b2, w_slot)
    r12 = _sc_gather(eout, jnp.concatenate([s1, s2]), 2 * S)
    out = _combine(x2, r12[:S], r12[S:])
    return out.reshape(1, S, D)


# speedup vs baseline: 1.0187x; 1.0187x over previous
import functools

import jax
import jax.numpy as jnp
from jax import lax
from jax.experimental import pallas as pl
from jax.experimental.pallas import tpu as pltpu
from jax.experimental.pallas import tpu_sc as plsc

S, D, H, HD = 2048, 2048, 16, 128
E, TOPK, FF = 8, 2, 1024
BM = 256
BN = 512
BT = 256
KB = 512
A = S * TOPK
PADDED = A + E * BT
NT = PADDED // BT
NW = 32


def _ln_qkv_body(x_ref, g_ref, b_ref, wq_ref, bq_ref, wk_ref, bk_ref,
                 wv_ref, bv_ref, q_ref, k_ref, v_ref):
    x = x_ref[...]
    m = jnp.mean(x, axis=-1, keepdims=True)
    var = jnp.mean((x - m) ** 2, axis=-1, keepdims=True)
    h = (x - m) / jnp.sqrt(var + 1e-5) * g_ref[...] + b_ref[...]
    q_ref[...] = jnp.dot(h, wq_ref[...], preferred_element_type=jnp.float32) + bq_ref[...]
    k_ref[...] = jnp.dot(h, wk_ref[...], preferred_element_type=jnp.float32) + bk_ref[...]
    v_ref[...] = jnp.dot(h, wv_ref[...], preferred_element_type=jnp.float32) + bv_ref[...]


def _ln_qkv(x, g, b, wq, bq, wk, bk, wv, bv):
    row = pl.BlockSpec((BM, D), lambda i, j: (i, 0))
    wsp = pl.BlockSpec((D, BN), lambda i, j: (0, j))
    bsp = pl.BlockSpec((1, BN), lambda i, j: (0, j))
    vec = pl.BlockSpec((1, D), lambda i, j: (0, 0))
    out = pl.BlockSpec((BM, BN), lambda i, j: (i, j))
    return pl.pallas_call(
        _ln_qkv_body,
        grid=(S // BM, D // BN),
        in_specs=[row, vec, vec, wsp, bsp, wsp, bsp, wsp, bsp],
        out_specs=[out, out, out],
        out_shape=[jax.ShapeDtypeStruct((S, D), jnp.float32)] * 3,
    )(x, g.reshape(1, D), b.reshape(1, D), wq, bq.reshape(1, D),
      wk, bk.reshape(1, D), wv, bv.reshape(1, D))


def _rot_half(x):
    return jnp.concatenate([-x[:, HD // 2:], x[:, : HD // 2]], axis=1)


NHP = 2


def _attn_body(q_ref, k_ref, v_ref, c_ref, s_ref, o_ref):
    i = pl.program_id(1)
    cq = c_ref[pl.ds(i * BM, BM), :]
    sq = s_ref[pl.ds(i * BM, BM), :]
    qq = q_ref[...]
    qs = [((qq[:, t * HD:(t + 1) * HD] * cq
            + _rot_half(qq[:, t * HD:(t + 1) * HD]) * sq).astype(jnp.bfloat16))
          for t in range(NHP)]

    def step(j, carry):
        kk = k_ref[pl.ds(j * KB, KB), :]
        ck = c_ref[pl.ds(j * KB, KB), :]
        sk = s_ref[pl.ds(j * KB, KB), :]
        vv = v_ref[pl.ds(j * KB, KB), :]
        rows = i * BM + lax.broadcasted_iota(jnp.int32, (BM, KB), 0)
        cols = j * KB + lax.broadcasted_iota(jnp.int32, (BM, KB), 1)
        causal = cols <= rows
        out = []
        for t in range(NHP):
            acc, m, l = carry[3 * t], carry[3 * t + 1], carry[3 * t + 2]
            kx = kk[:, t * HD:(t + 1) * HD]
            kj = (kx * ck + _rot_half(kx) * sk).astype(jnp.bfloat16)
            vj = vv[:, t * HD:(t + 1) * HD].astype(jnp.bfloat16)
            s = lax.dot_general(qs[t], kj, (((1,), (1,)), ((), ())),
                                preferred_element_type=jnp.float32)
            s = s * (1.0 / jnp.sqrt(jnp.float32(HD)))
            s = jnp.where(causal, s, jnp.float32(-1e9))
            m_new = jnp.maximum(m, jnp.max(s, axis=1, keepdims=True))
            p = jnp.exp(s - m_new)
            corr = jnp.exp(m - m_new)
            l = l * corr + jnp.sum(p, axis=1, keepdims=True)
            acc = acc * corr + jnp.dot(p.astype(jnp.bfloat16), vj,
                                       preferred_element_type=jnp.float32)
            out += [acc, m_new, l]
        return tuple(out)

    z = jnp.zeros((BM, HD), jnp.float32)
    mneg = jnp.full((BM, 1), -1e30, jnp.float32)
    zl = jnp.zeros((BM, 1), jnp.float32)
    res = lax.fori_loop(0, (i * BM) // KB + 1, step, (z, mneg, zl) * NHP)
    o_ref[...] = jnp.concatenate(
        [res[3 * t] / res[3 * t + 2] for t in range(NHP)], axis=1)


def _attention(q, k, v, cos, sin):
    qsp = pl.BlockSpec((BM, NHP * HD), lambda h, i: (i, h))
    ksp = pl.BlockSpec((S, NHP * HD), lambda h, i: (0, h))
    csp = pl.BlockSpec((S, HD), lambda h, i: (0, 0))
    return pl.pallas_call(
        _attn_body,
        grid=(H // NHP, S // BM),
        in_specs=[qsp, ksp, ksp, csp, csp],
        out_specs=qsp,
        out_shape=jax.ShapeDtypeStruct((S, D), jnp.float32),
    )(q, k, v, cos, sin)


def _oproj_body(a_ref, w_ref, b_ref, r_ref, o_ref):
    o_ref[...] = (jnp.dot(a_ref[...], w_ref[...], preferred_element_type=jnp.float32)
                  + b_ref[...] + r_ref[...])


def _oproj_res(attn, wo, bo, res):
    return pl.pallas_call(
        _oproj_body,
        grid=(S // BM, D // BN),
        in_specs=[pl.BlockSpec((BM, D), lambda i, j: (i, 0)),
                  pl.BlockSpec((D, BN), lambda i, j: (0, j)),
                  pl.BlockSpec((1, BN), lambda i, j: (0, j)),
                  pl.BlockSpec((BM, BN), lambda i, j: (i, j))],
        out_specs=pl.BlockSpec((BM, BN), lambda i, j: (i, j)),
        out_shape=jax.ShapeDtypeStruct((S, D), jnp.float32),
    )(attn, wo, bo.reshape(1, D), res)


def _gate_body(x_ref, g_ref, b_ref, wg_ref, h_ref, mi_ref, mf_ref):
    x = x_ref[...]
    m = jnp.mean(x, axis=-1, keepdims=True)
    var = jnp.mean((x - m) ** 2, axis=-1, keepdims=True)
    h = (x - m) / jnp.sqrt(var + 1e-5) * g_ref[...] + b_ref[...]
    h_ref[...] = h
    lg = jnp.dot(h, wg_ref[...], preferred_element_type=jnp.float32)
    col = lax.broadcasted_iota(jnp.int32, lg.shape, 1)
    neg = jnp.float32(-1e30)
    lg = jnp.where(col < E, lg, neg)
    m1 = jnp.max(lg, axis=1, keepdims=True)
    i1 = jnp.min(jnp.where(lg >= m1, col, 128), axis=1, keepdims=True)
    lg2 = jnp.where(col == i1, neg, lg)
    m2 = jnp.max(lg2, axis=1, keepdims=True)
    i2 = jnp.min(jnp.where(lg2 >= m2, col, 128), axis=1, keepdims=True)
    w1 = 1.0 / (1.0 + jnp.exp(m2 - m1))
    w2 = 1.0 - w1
    mi_ref[...] = jnp.where(col == 0, i1, jnp.where(col == 1, i2, 0))
    mf_ref[...] = jnp.where(col == 0, w1, jnp.where(col == 1, w2, 0.0))


def _ln_gate(x, g, b, wg_pad):
    row = pl.BlockSpec((BM, D), lambda i: (i, 0))
    vec = pl.BlockSpec((1, D), lambda i: (0, 0))
    meta = pl.BlockSpec((BM, 128), lambda i: (i, 0))
    return pl.pallas_call(
        _gate_body,
        grid=(S // BM,),
        in_specs=[row, vec, vec, pl.BlockSpec((D, 128), lambda i: (0, 0))],
        out_specs=[row, meta, meta],
        out_shape=[jax.ShapeDtypeStruct((S, D), jnp.float32),
                   jax.ShapeDtypeStruct((S, 128), jnp.int32),
                   jax.ShapeDtypeStruct((S, 128), jnp.float32)],
    )(x, g.reshape(1, D), b.reshape(1, D), wg_pad)


def _sc_gather(table, idx, n_rows, chunk=16, nbuf=3):
    per_w = n_rows // NW
    n_chunks = per_w // chunk
    mesh = plsc.VectorSubcoreMesh(core_axis_name="c", subcore_axis_name="s")

    @functools.partial(
        pl.kernel, mesh=mesh,
        out_type=jax.ShapeDtypeStruct((n_rows, D), jnp.float32),
        scratch_types=[pltpu.VMEM((per_w,), jnp.int32),
                       pltpu.VMEM((nbuf, chunk, D), jnp.float32),
                       [pltpu.SemaphoreType.DMA] * nbuf,
                       [pltpu.SemaphoreType.DMA] * nbuf],
    )
    def k(table_hbm, idx_hbm, out_hbm, idx_v, rows_v, gsem, osem):
        wid = lax.axis_index("s") * 2 + lax.axis_index("c")
        base = wid * per_w
        pltpu.sync_copy(idx_hbm.at[pl.ds(base, per_w)], idx_v)
        gd = [None] * n_chunks
        od = [None] * n_chunks

        def start_gather(c):
            b = c % nbuf
            gd[c] = pltpu.async_copy(
                table_hbm.at[idx_v.at[pl.ds(c * chunk, chunk)]], rows_v.at[b], gsem[b])

        for c in range(min(nbuf, n_chunks)):
            start_gather(c)
        for c in range(n_chunks):
            b = c % nbuf
            gd[c].wait()
            od[c] = pltpu.async_copy(
                rows_v.at[b], out_hbm.at[pl.ds(base + c * chunk, chunk)], osem[b])
            if c + nbuf < n_chunks:
                od[c].wait()
                start_gather(c + nbuf)
        for c in range(max(0, n_chunks - nbuf), n_chunks):
            od[c].wait()

    return k(table, idx)


def _ffn_body(te_ref, used_ref, xs_ref, w1_ref, b1_ref, w2_ref, b2_ref,
              ws_ref, o_ref):
    i = pl.program_id(0)

    @pl.when(used_ref[i] > 0)
    def _compute():
        x = xs_ref[...].astype(jnp.bfloat16)
        h = jnp.dot(x, w1_ref[0].astype(jnp.bfloat16),
                    preferred_element_type=jnp.float32) + b1_ref[0]
        h = h * 0.5 * (1.0 + lax.erf(h * (1.0 / jnp.sqrt(jnp.float32(2.0)))))
        o = jnp.dot(h.astype(jnp.bfloat16), w2_ref[0].astype(jnp.bfloat16),
                    preferred_element_type=jnp.float32) + b2_ref[0]
        o_ref[...] = o * ws_ref[...]


def _grouped_ffn(xs, tile_expert, tile_used, w1, b1, w2, b2, w_slot):
    grid_spec = pltpu.PrefetchScalarGridSpec(
        num_scalar_prefetch=2,
        grid=(NT,),
        in_specs=[
            pl.BlockSpec((BT, D), lambda i, te, tu: (i, 0)),
            pl.BlockSpec((1, D, FF), lambda i, te, tu: (te[i], 0, 0)),
            pl.BlockSpec((1, 1, FF), lambda i, te, tu: (te[i], 0, 0)),
            pl.BlockSpec((1, FF, D), lambda i, te, tu: (te[i], 0, 0)),
            pl.BlockSpec((1, 1, D), lambda i, te, tu: (te[i], 0, 0)),
            pl.BlockSpec((BT, 1), lambda i, te, tu: (i, 0)),
        ],
        out_specs=pl.BlockSpec((BT, D), lambda i, te, tu: (i, 0)),
    )
    return pl.pallas_call(
        _ffn_body,
        grid_spec=grid_spec,
        out_shape=jax.ShapeDtypeStruct((PADDED, D), jnp.float32),
    )(tile_expert, tile_used, xs, w1,
      b1.reshape(E, 1, FF), w2, b2.reshape(E, 1, D),
      w_slot.reshape(PADDED, 1))


def _combine_body(r_ref, a_ref, b_ref, o_ref):
    o_ref[...] = r_ref[...] + a_ref[...] + b_ref[...]


def _combine(res, r1, r2):
    blk = pl.BlockSpec((BM, BN), lambda i, j: (i, j))
    return pl.pallas_call(
        _combine_body,
        grid=(S // BM, D // BN),
        in_specs=[blk, blk, blk],
        out_specs=blk,
        out_shape=jax.ShapeDtypeStruct((S, D), jnp.float32),
    )(res, r1, r2)


def kernel(hidden_states, wq, bq, wk, bk, wv, bv, wo, bo,
           ln1_g, ln1_b, ln2_g, ln2_b, w_gate, w1, b1, w2, b2):
    x = hidden_states.reshape(S, D)

    pos = jnp.arange(S, dtype=jnp.float32)
    inv_freq = 1.0 / (10000.0 ** (jnp.arange(0, HD, 2, dtype=jnp.float32) / HD))
    freqs = pos[:, None] * inv_freq[None, :]
    emb = jnp.concatenate((freqs, freqs), axis=-1)
    cos = jnp.cos(emb)
    sin = jnp.sin(emb)

    q, k, v = _ln_qkv(x, ln1_g, ln1_b, wq, bq, wk, bk, wv, bv)
    attn = _attention(q, k, v, cos, sin)
    x2 = _oproj_res(attn, wo, bo, x)

    wg_pad = jnp.pad(w_gate, ((0, 0), (0, 128 - E)))
    h2, mi, mf = _ln_gate(x2, ln2_g, ln2_b, wg_pad)
    topi = mi[:, :TOPK]
    wts = mf[:, :TOPK]

    ef = topi.reshape(-1)
    onehot = (ef[:, None] == jnp.arange(E, dtype=ef.dtype)[None, :]).astype(jnp.int32)
    counts = jnp.sum(onehot, axis=0)
    pc = ((counts + BT - 1) // BT) * BT
    ends_p = jnp.cumsum(pc)
    starts_p = ends_p - pc
    rank = jnp.take_along_axis(jnp.cumsum(onehot, axis=0), ef[:, None], axis=1)[:, 0] - 1
    slot_of_a = (starts_p[ef] + rank).astype(jnp.int32)
    s1 = slot_of_a[0::2]
    s2 = slot_of_a[1::2]
    tok = jnp.arange(A, dtype=jnp.int32) // TOPK
    ri = (jnp.arange(PADDED, dtype=jnp.int32) % S).at[slot_of_a].set(tok)
    w_slot = jnp.zeros((PADDED,), jnp.float32).at[slot_of_a].set(wts.reshape(-1))
    tile_start = jnp.arange(NT, dtype=jnp.int32) * BT
    tile_used = (tile_start < ends_p[-1]).astype(jnp.int32)
    tile_expert = jnp.minimum(
        jnp.searchsorted(ends_p, tile_start, side="right"), E - 1).astype(jnp.int32)

    xs = _sc_gather(h2, ri, PADDED)
    eout = _grouped_ffn(xs, tile_expert, tile_used, w1, b1, w2, b2, w_slot)
    r12 = _sc_gather(eout, jnp.concatenate([s1, s2]), 2 * S)
    out = _combine(x2, r12[:S], r12[S:])
    return out.reshape(1, S, D)
